# baseline (speedup 1.0000x reference)
import functools

import jax
import jax.numpy as jnp
from jax import lax
from jax.experimental import pallas as pl
from jax.experimental.pallas import tpu as pltpu
from jax.experimental.pallas import tpu_sc as plsc

N = 10000
E = 320000
H = 128
L = 7
G = 64
VOCAB = 8
EPS = 1e-7

_BLK = 1000

_NSC = 2
_NSUB = 16
_NW = _NSC * _NSUB
_CSZ = 128
_CH = 80
_NBUF = 5
_NGRP = _CH // _NBUF
_NGT = _NW * _NGRP + 1
_HALF = 5056
_MROW = 5120
_RPT = _MROW // _NSUB


def _mp_body(xp_hbm, gidx_hbm, didx_hbm, zeros_hbm, out_hbm,
             m_sh, gib0, gib1, dib0, dib1, r0, r1, r2, r3, r4,
             gs0, gs1, gs2, gs3, gs4, ss0, ss1, ss2, ss3, ss4, is0, is1):
    c = lax.axis_index("c")
    s = lax.axis_index("s")
    wid = c * _NSUB + s
    gbase = wid * _NGRP
    bufs = (r0, r1, r2, r3, r4)
    gsems = (gs0, gs1, gs2, gs3, gs4)
    ssems = (ss0, ss1, ss2, ss3, ss4)
    gibs = (gib0, gib1)
    dibs = (dib0, dib1)
    isems = (is0, is1)

    def _idx_load(g, p):
        pltpu.async_copy(gidx_hbm.at[gbase + g], gibs[p], isems[p])
        pltpu.async_copy(didx_hbm.at[gbase + g], dibs[p], isems[p])

    def _idx_wait(p):
        pltpu.make_async_copy(gidx_hbm.at[0], gibs[p], isems[p]).wait()
        pltpu.make_async_copy(didx_hbm.at[0], dibs[p], isems[p]).wait()

    def _start_gather(p, b):
        pltpu.async_copy(xp_hbm.at[gibs[p].at[b]], bufs[b], gsems[b])

    def _wait_gather(p, b):
        pltpu.make_async_copy(xp_hbm.at[gibs[p].at[b]], bufs[b],
                              gsems[b]).wait()

    def _start_scatter(p, b):
        pltpu.async_copy(bufs[b], m_sh.at[dibs[p].at[b]], ssems[b], add=True)

    def _wait_scatter(p, b):
        pltpu.make_async_copy(bufs[b], m_sh.at[dibs[p].at[b]],
                              ssems[b]).wait()

    pltpu.sync_copy(zeros_hbm.at[pl.ds(s * _RPT, _RPT)],
                    m_sh.at[pl.ds(s * _RPT, _RPT)])

    _idx_load(0, 0)
    _idx_load(1, 1)
    plsc.subcore_barrier()
    _idx_wait(0)
    for b in range(_NBUF):
        _start_gather(0, b)

    def _do_group(p, q, prefetch_g):
        for b in range(_NBUF):
            _wait_gather(p, b)
            _start_scatter(p, b)
        _idx_wait(q)
        for b in range(_NBUF):
            _wait_scatter(p, b)
            _start_gather(q, b)
        _idx_load(prefetch_g, p)

    def _pair(i, carry):
        g = 2 * i
        _do_group(0, 1, g + 2)
        _do_group(1, 0, g + 3)
        return carry

    lax.fori_loop(0, _NGRP // 2 - 1, _pair, 0)
    _do_group(0, 1, _NGRP)
    for b in range(_NBUF):
        _wait_gather(1, b)
        _start_scatter(1, b)
    for b in range(_NBUF):
        _wait_scatter(1, b)
    _idx_wait(0)

    plsc.subcore_barrier()
    pltpu.sync_copy(m_sh.at[pl.ds(s * _RPT, _RPT)],
                    out_hbm.at[c, pl.ds(s * _RPT, _RPT)])


_mp_call = pl.kernel(
    _mp_body,
    out_type=jax.ShapeDtypeStruct((_NSC, _MROW, H), jnp.float32),
    mesh=plsc.VectorSubcoreMesh(core_axis_name="c", subcore_axis_name="s"),
    scratch_types=[
        pltpu.VMEM_SHARED((_MROW, H), jnp.float32),
        pltpu.VMEM((_NBUF, _CSZ), jnp.int32),
        pltpu.VMEM((_NBUF, _CSZ), jnp.int32),
        pltpu.VMEM((_NBUF, _CSZ), jnp.int32),
        pltpu.VMEM((_NBUF, _CSZ), jnp.int32),
        pltpu.VMEM((_CSZ, H), jnp.float32),
        pltpu.VMEM((_CSZ, H), jnp.float32),
        pltpu.VMEM((_CSZ, H), jnp.float32),
        pltpu.VMEM((_CSZ, H), jnp.float32),
        pltpu.VMEM((_CSZ, H), jnp.float32),
    ] + [pltpu.SemaphoreType.DMA] * 12,
)


def _enc_body(x_ref, w_ref, b_ref, out_ref):
    out_ref[...] = x_ref[...] @ w_ref[...] + b_ref[...]


def _encode(x, W_enc, b_enc):
    return pl.pallas_call(
        _enc_body,
        grid=(N // _BLK,),
        in_specs=[
            pl.BlockSpec((_BLK, H), lambda i: (i, 0)),
            pl.BlockSpec((H, H), lambda i: (0, 0)),
            pl.BlockSpec((1, H), lambda i: (0, 0)),
        ],
        out_specs=pl.BlockSpec((_BLK, H), lambda i: (i, 0)),
        out_shape=jax.ShapeDtypeStruct((N, H), jnp.float32),
    )(x, W_enc, b_enc.reshape(1, H))


def _xp_body(h2_ref, table_ref, out_ref):
    h2 = h2_ref[...]
    for a in range(VOCAB):
        out_ref[a] = jnp.maximum(h2 + table_ref[a], 0.0) + EPS


def _xprime(h2, table):
    out = pl.pallas_call(
        _xp_body,
        grid=(N // _BLK,),
        in_specs=[
            pl.BlockSpec((_BLK, H), lambda i: (i, 0)),
            pl.BlockSpec((VOCAB, H), lambda i: (0, 0)),
        ],
        out_specs=pl.BlockSpec((VOCAB, _BLK, H), lambda i: (0, i, 0)),
        out_shape=jax.ShapeDtypeStruct((VOCAB, N, H), jnp.float32),
    )(h2, table)
    return out.reshape(VOCAB * N, H)


def _mlp_body(h2_ref, m_ref, res_ref, w1_ref, b1_ref, s1_ref, bb1_ref,
              w2_ref, b2_ref, out_ref):
    hsum = h2_ref[...] + m_ref[...]
    t = hsum @ w1_ref[...] + b1_ref[...]
    mu = jnp.mean(t, axis=-1, keepdims=True)
    var = jnp.mean((t - mu) ** 2, axis=-1, keepdims=True)
    t = (t - mu) / jnp.sqrt(var + 1e-5) * s1_ref[...] + bb1_ref[...]
    t = jnp.maximum(t, 0.0)
    out_ref[...] = t @ w2_ref[...] + b2_ref[...] + res_ref[...]


def _mlp(h2, m, res, W1l, b1l, s1l, bb1l, W2l, b2l):
    return pl.pallas_call(
        _mlp_body,
        grid=(N // _BLK,),
        in_specs=[
            pl.BlockSpec((_BLK, H), lambda i: (i, 0)),
            pl.BlockSpec((_BLK, H), lambda i: (i, 0)),
            pl.BlockSpec((_BLK, H), lambda i: (i, 0)),
            pl.BlockSpec((H, 2 * H), lambda i: (0, 0)),
            pl.BlockSpec((1, 2 * H), lambda i: (0, 0)),
            pl.BlockSpec((1, 2 * H), lambda i: (0, 0)),
            pl.BlockSpec((1, 2 * H), lambda i: (0, 0)),
            pl.BlockSpec((2 * H, H), lambda i: (0, 0)),
            pl.BlockSpec((1, H), lambda i: (0, 0)),
        ],
        out_specs=pl.BlockSpec((_BLK, H), lambda i: (i, 0)),
        out_shape=jax.ShapeDtypeStruct((N, H), jnp.float32),
    )(h2, m, res, W1l, b1l.reshape(1, -1), s1l.reshape(1, -1),
      bb1l.reshape(1, -1), W2l, b2l.reshape(1, -1))


def _layernorm(x, s, b):
    mu = jnp.mean(x, axis=-1, keepdims=True)
    var = jnp.var(x, axis=-1, keepdims=True)
    return (x - mu) / jnp.sqrt(var + 1e-5) * s + b


def kernel(x, edge_index, edge_attr, batch, W_enc, b_enc, edge_table, W1, b1,
           ln1s, ln1b, W2, b2, norm_s, norm_b, W_pred, b_pred):
    src = edge_index[0]
    dst = edge_index[1]

    total = _NGT * _NBUF * _CSZ
    half_slots = _NSUB * _CH * _CSZ
    cidx = edge_attr.astype(jnp.int32) * N + src
    m1 = dst >= _HALF
    c1 = jnp.cumsum(m1.astype(jnp.int32))
    rank0 = jnp.arange(E, dtype=jnp.int32) + 1 - c1
    pos = jnp.where(m1, half_slots + c1, rank0) - 1
    pos = jnp.where(jnp.where(m1, c1, rank0) <= half_slots, pos, total)
    ldst = dst - jnp.where(m1, _HALF, 0).astype(dst.dtype)
    gidx = jnp.zeros((total,), jnp.int32).at[pos].set(
        cidx, unique_indices=True, mode="drop").reshape(_NGT, _NBUF, _CSZ)
    didx = jnp.full((total,), _HALF, jnp.int32).at[pos].set(
        ldst.astype(jnp.int32), unique_indices=True,
        mode="drop").reshape(_NGT, _NBUF, _CSZ)
    zeros = jnp.zeros((_MROW, H), jnp.float32)

    h = _encode(x, W_enc, b_enc)
    for l in range(L):
        if l == 0:
            h2 = h
            res = jnp.zeros((N, H), jnp.float32)
        else:
            h2 = jax.nn.relu(_layernorm(h, norm_s[l - 1], norm_b[l - 1]))
            res = h
        xp = _xprime(h2, edge_table)
        parts = _mp_call(xp, gidx, didx, zeros)
        m = jnp.concatenate([parts[0, :_HALF], parts[1, :N - _HALF]], axis=0)
        h = _mlp(h2, m, res, W1[l], b1[l], ln1s[l], ln1b[l], W2[l], b2[l])

    hf = _layernorm(h, norm_s[L - 1], norm_b[L - 1])
    sums = jax.ops.segment_sum(hf, batch, num_segments=G)
    counts = jax.ops.segment_sum(jnp.ones((N,), jnp.float32), batch,
                                 num_segments=G)
    hg = sums / jnp.maximum(counts, 1.0)[:, None]
    out = jax.nn.sigmoid(hg @ W_pred + b_pred)
    return out.reshape(-1)


# speedup vs baseline: 3.3244x; 3.3244x over previous
import functools

import jax
import jax.numpy as jnp
from jax import lax
from jax.experimental import pallas as pl
from jax.experimental.pallas import tpu as pltpu
from jax.experimental.pallas import tpu_sc as plsc

N = 10000
E = 320000
H = 128
L = 7
G = 64
VOCAB = 8
EPS = 1e-7

_BLK = 1000

_NSC = 2
_NSUB = 16
_NW = _NSC * _NSUB
_CSZ = 120
_CH = 84
_NBUF = 3
_NGRP = _CH // _NBUF
_NGT = _NW * _NGRP + 1
_NPAD = 10112
_RPT = _NPAD // _NSUB


def _mp_body(xp_hbm, gidx_hbm, didx_hbm, zeros_hbm, out_hbm,
             m_sh, gib0, gib1, dib0, dib1, r0, r1, r2,
             gs0, gs1, gs2, ss0, ss1, ss2, is0, is1):
    c = lax.axis_index("c")
    s = lax.axis_index("s")
    wid = c * _NSUB + s
    gbase = wid * _NGRP
    bufs = (r0, r1, r2)
    gsems = (gs0, gs1, gs2)
    ssems = (ss0, ss1, ss2)
    gibs = (gib0, gib1)
    dibs = (dib0, dib1)
    isems = (is0, is1)

    def _idx_load(g, p):
        pltpu.async_copy(gidx_hbm.at[gbase + g], gibs[p], isems[p])
        pltpu.async_copy(didx_hbm.at[gbase + g], dibs[p], isems[p])

    def _idx_wait(p):
        pltpu.make_async_copy(gidx_hbm.at[0], gibs[p], isems[p]).wait()
        pltpu.make_async_copy(didx_hbm.at[0], dibs[p], isems[p]).wait()

    def _start_gather(p, b):
        pltpu.async_copy(xp_hbm.at[gibs[p].at[b]], bufs[b], gsems[b])

    def _wait_gather(p, b):
        pltpu.make_async_copy(xp_hbm.at[gibs[p].at[b]], bufs[b],
                              gsems[b]).wait()

    def _start_scatter(p, b):
        pltpu.async_copy(bufs[b], m_sh.at[dibs[p].at[b]], ssems[b], add=True)

    def _wait_scatter(p, b):
        pltpu.make_async_copy(bufs[b], m_sh.at[dibs[p].at[b]],
                              ssems[b]).wait()

    pltpu.sync_copy(zeros_hbm.at[pl.ds(s * _RPT, _RPT)],
                    m_sh.at[pl.ds(s * _RPT, _RPT)])

    _idx_load(0, 0)
    _idx_load(1, 1)
    plsc.subcore_barrier()
    _idx_wait(0)
    for b in range(_NBUF):
        _start_gather(0, b)

    def _do_group(p, q, prefetch_g):
        for b in range(_NBUF):
            _wait_gather(p, b)
            _start_scatter(p, b)
        _idx_wait(q)
        for b in range(_NBUF):
            _wait_scatter(p, b)
            _start_gather(q, b)
        _idx_load(prefetch_g, p)

    def _pair(i, carry):
        g = 2 * i
        _do_group(0, 1, g + 2)
        _do_group(1, 0, g + 3)
        return carry

    lax.fori_loop(0, _NGRP // 2 - 1, _pair, 0)
    _do_group(0, 1, _NGRP)
    for b in range(_NBUF):
        _wait_gather(1, b)
        _start_scatter(1, b)
    for b in range(_NBUF):
        _wait_scatter(1, b)
    _idx_wait(0)

    plsc.subcore_barrier()
    pltpu.sync_copy(m_sh.at[pl.ds(s * _RPT, _RPT)],
                    out_hbm.at[c, pl.ds(s * _RPT, _RPT)])


_mp_call = pl.kernel(
    _mp_body,
    out_type=jax.ShapeDtypeStruct((_NSC, _NPAD, H), jnp.float32),
    mesh=plsc.VectorSubcoreMesh(core_axis_name="c", subcore_axis_name="s"),
    scratch_types=[
        pltpu.VMEM_SHARED((_NPAD, H), jnp.float32),
        pltpu.VMEM((_NBUF, _CSZ), jnp.int32),
        pltpu.VMEM((_NBUF, _CSZ), jnp.int32),
        pltpu.VMEM((_NBUF, _CSZ), jnp.int32),
        pltpu.VMEM((_NBUF, _CSZ), jnp.int32),
        pltpu.VMEM((_CSZ, H), jnp.float32),
        pltpu.VMEM((_CSZ, H), jnp.float32),
        pltpu.VMEM((_CSZ, H), jnp.float32),
    ] + [pltpu.SemaphoreType.DMA] * 8,
)


def _enc_body(x_ref, w_ref, b_ref, table_ref, h_ref, xp_ref):
    h = x_ref[...] @ w_ref[...] + b_ref[...]
    h_ref[...] = h
    for a in range(VOCAB):
        xp_ref[a] = jnp.maximum(h + table_ref[a], 0.0) + EPS


def _encode(x, W_enc, b_enc, table):
    h, xp = pl.pallas_call(
        _enc_body,
        grid=(N // _BLK,),
        in_specs=[
            pl.BlockSpec((_BLK, H), lambda i: (i, 0)),
            pl.BlockSpec((H, H), lambda i: (0, 0)),
            pl.BlockSpec((1, H), lambda i: (0, 0)),
            pl.BlockSpec((VOCAB, H), lambda i: (0, 0)),
        ],
        out_specs=[
            pl.BlockSpec((_BLK, H), lambda i: (i, 0)),
            pl.BlockSpec((VOCAB, _BLK, H), lambda i: (0, i, 0)),
        ],
        out_shape=[
            jax.ShapeDtypeStruct((N, H), jnp.float32),
            jax.ShapeDtypeStruct((VOCAB, N, H), jnp.float32),
        ],
    )(x, W_enc, b_enc.reshape(1, H), table)
    return h, xp.reshape(VOCAB * N, H)


def _ln(t, s, b):
    mu = jnp.mean(t, axis=-1, keepdims=True)
    var = jnp.mean((t - mu) ** 2, axis=-1, keepdims=True)
    return (t - mu) / jnp.sqrt(var + 1e-5) * s + b


def _mlp_body(h2_ref, parts_ref, res_ref, w1_ref, b1_ref, s1_ref, bb1_ref,
              w2_ref, b2_ref, ns_ref, nb_ref, table_ref,
              h_ref, h2n_ref, xp_ref):
    t = (h2_ref[...] + parts_ref[0] + parts_ref[1]) @ w1_ref[...] + b1_ref[...]
    t = _ln(t, s1_ref[...], bb1_ref[...])
    t = jnp.maximum(t, 0.0)
    h = t @ w2_ref[...] + b2_ref[...] + res_ref[...]
    h_ref[...] = h
    h2n = jnp.maximum(_ln(h, ns_ref[...], nb_ref[...]), 0.0)
    h2n_ref[...] = h2n
    for a in range(VOCAB):
        xp_ref[a] = jnp.maximum(h2n + table_ref[a], 0.0) + EPS


def _mlp_x(h2, parts, res, W1l, b1l, s1l, bb1l, W2l, b2l, nsl, nbl, table):
    h, h2n, xp = pl.pallas_call(
        _mlp_body,
        grid=(N // _BLK,),
        in_specs=[
            pl.BlockSpec((_BLK, H), lambda i: (i, 0)),
            pl.BlockSpec((_NSC, _BLK, H), lambda i: (0, i, 0)),
            pl.BlockSpec((_BLK, H), lambda i: (i, 0)),
            pl.BlockSpec((H, 2 * H), lambda i: (0, 0)),
            pl.BlockSpec((1, 2 * H), lambda i: (0, 0)),
            pl.BlockSpec((1, 2 * H), lambda i: (0, 0)),
            pl.BlockSpec((1, 2 * H), lambda i: (0, 0)),
            pl.BlockSpec((2 * H, H), lambda i: (0, 0)),
            pl.BlockSpec((1, H), lambda i: (0, 0)),
            pl.BlockSpec((1, H), lambda i: (0, 0)),
            pl.BlockSpec((1, H), lambda i: (0, 0)),
            pl.BlockSpec((VOCAB, H), lambda i: (0, 0)),
        ],
        out_specs=[
            pl.BlockSpec((_BLK, H), lambda i: (i, 0)),
            pl.BlockSpec((_BLK, H), lambda i: (i, 0)),
            pl.BlockSpec((VOCAB, _BLK, H), lambda i: (0, i, 0)),
        ],
        out_shape=[
            jax.ShapeDtypeStruct((N, H), jnp.float32),
            jax.ShapeDtypeStruct((N, H), jnp.float32),
            jax.ShapeDtypeStruct((VOCAB, N, H), jnp.float32),
        ],
    )(h2, parts, res, W1l, b1l.reshape(1, -1), s1l.reshape(1, -1),
      bb1l.reshape(1, -1), W2l, b2l.reshape(1, -1), nsl.reshape(1, -1),
      nbl.reshape(1, -1), table)
    return h, h2n, xp.reshape(VOCAB * N, H)


def _fin_body(h2_ref, parts_ref, res_ref, w1_ref, b1_ref, s1_ref, bb1_ref,
              w2_ref, b2_ref, ns_ref, nb_ref, p_ref, wp_ref, bp_ref,
              out_ref, sums_ref, cnt_ref):
    i = pl.program_id(0)
    t = (h2_ref[...] + parts_ref[0] + parts_ref[1]) @ w1_ref[...] + b1_ref[...]
    t = _ln(t, s1_ref[...], bb1_ref[...])
    t = jnp.maximum(t, 0.0)
    h = t @ w2_ref[...] + b2_ref[...] + res_ref[...]
    hf = _ln(h, ns_ref[...], nb_ref[...])
    p = p_ref[...]
    psum = lax.dot_general(p, hf, (((0,), (0,)), ((), ())),
                           preferred_element_type=jnp.float32)
    pcnt = lax.dot_general(p, jnp.ones((_BLK, 1), jnp.float32),
                           (((0,), (0,)), ((), ())),
                           preferred_element_type=jnp.float32)

    @pl.when(i == 0)
    def _():
        sums_ref[...] = jnp.zeros_like(sums_ref)
        cnt_ref[...] = jnp.zeros_like(cnt_ref)

    sums_ref[...] += psum
    cnt_ref[...] += pcnt

    @pl.when(i == N // _BLK - 1)
    def _():
        hg = sums_ref[...] / jnp.maximum(cnt_ref[...], 1.0)
        out_ref[...] = jax.nn.sigmoid(hg @ wp_ref[...] + bp_ref[...])


def _final(h2, parts, res, W1l, b1l, s1l, bb1l, W2l, b2l, nsl, nbl, P,
           W_pred, b_pred):
    return pl.pallas_call(
        _fin_body,
        grid=(N // _BLK,),
        in_specs=[
            pl.BlockSpec((_BLK, H), lambda i: (i, 0)),
            pl.BlockSpec((_NSC, _BLK, H), lambda i: (0, i, 0)),
            pl.BlockSpec((_BLK, H), lambda i: (i, 0)),
            pl.BlockSpec((H, 2 * H), lambda i: (0, 0)),
            pl.BlockSpec((1, 2 * H), lambda i: (0, 0)),
            pl.BlockSpec((1, 2 * H), lambda i: (0, 0)),
            pl.BlockSpec((1, 2 * H), lambda i: (0, 0)),
            pl.BlockSpec((2 * H, H), lambda i: (0, 0)),
            pl.BlockSpec((1, H), lambda i: (0, 0)),
            pl.BlockSpec((1, H), lambda i: (0, 0)),
            pl.BlockSpec((1, H), lambda i: (0, 0)),
            pl.BlockSpec((_BLK, G), lambda i: (i, 0)),
            pl.BlockSpec((H, 1), lambda i: (0, 0)),
            pl.BlockSpec((1, 1), lambda i: (0, 0)),
        ],
        out_specs=pl.BlockSpec((G, 1), lambda i: (0, 0)),
        out_shape=jax.ShapeDtypeStruct((G, 1), jnp.float32),
        scratch_shapes=[
            pltpu.VMEM((G, H), jnp.float32),
            pltpu.VMEM((G, 1), jnp.float32),
        ],
    )(h2, parts, res, W1l, b1l.reshape(1, -1), s1l.reshape(1, -1),
      bb1l.reshape(1, -1), W2l, b2l.reshape(1, -1), nsl.reshape(1, -1),
      nbl.reshape(1, -1), P, W_pred, b_pred.reshape(1, 1))


def kernel(x, edge_index, edge_attr, batch, W_enc, b_enc, edge_table, W1, b1,
           ln1s, ln1b, W2, b2, norm_s, norm_b, W_pred, b_pred):
    src = edge_index[0]
    dst = edge_index[1]

    total = _NGT * _NBUF * _CSZ
    npads = total - E
    cidx = edge_attr.astype(jnp.int32) * N + src
    gidx = jnp.concatenate(
        [cidx, jnp.zeros((npads,), jnp.int32)]).reshape(_NGT, _NBUF, _CSZ)
    trash = N + (jnp.arange(npads, dtype=jnp.int32) % (_NPAD - N))
    didx = jnp.concatenate([dst, trash]).reshape(_NGT, _NBUF, _CSZ)
    zeros = jnp.zeros((_NPAD, H), jnp.float32)
    P = (batch[:, None] == jnp.arange(G, dtype=batch.dtype)[None, :]
         ).astype(jnp.float32)

    h, xp = _encode(x, W_enc, b_enc, edge_table)
    h2 = h
    res = jnp.zeros((N, H), jnp.float32)
    for l in range(L - 1):
        parts = _mp_call(xp, gidx, didx, zeros)
        h, h2, xp = _mlp_x(h2, parts, res, W1[l], b1[l], ln1s[l], ln1b[l],
                           W2[l], b2[l], norm_s[l], norm_b[l], edge_table)
        res = h
    parts = _mp_call(xp, gidx, didx, zeros)
    out = _final(h2, parts, res, W1[L - 1], b1[L - 1], ln1s[L - 1],
                 ln1b[L - 1], W2[L - 1], b2[L - 1], norm_s[L - 1],
                 norm_b[L - 1], P, W_pred, b_pred)
    return out.reshape(-1)


# speedup vs baseline: 4.7271x; 1.4219x over previous
import functools

import jax
import jax.numpy as jnp
from jax import lax
from jax.experimental import pallas as pl
from jax.experimental.pallas import tpu as pltpu
from jax.experimental.pallas import tpu_sc as plsc

N = 10000
E = 320000
H = 128
L = 7
G = 64
VOCAB = 8
EPS = 1e-7

_BLK = 1000

_NSC = 2
_NSUB = 16
_NW = _NSC * _NSUB
_CSZ = 120
_NBUF = 3
_NGRP0 = 36
_NGRP1 = 20
_NGT = _NSUB * (_NGRP0 + _NGRP1) + 2
_NPAD = 10112
_RPT = _NPAD // _NSUB


def _mp_body(xp_hbm, gidx_hbm, didx_hbm, zeros_hbm, out_hbm,
             m_sh, gib0, gib1, dib0, dib1, r0, r1, r2,
             gs0, gs1, gs2, ss0, ss1, ss2, is0, is1):
    c = lax.axis_index("c")
    s = lax.axis_index("s")
    wid = c * _NSUB + s
    gbase = jnp.where(c == 0, s * _NGRP0, _NSUB * _NGRP0 + s * _NGRP1)
    npair = jnp.where(c == 0, _NGRP0 // 2, _NGRP1 // 2)
    bufs = (r0, r1, r2)
    gsems = (gs0, gs1, gs2)
    ssems = (ss0, ss1, ss2)
    gibs = (gib0, gib1)
    dibs = (dib0, dib1)
    isems = (is0, is1)

    def _idx_load(g, p):
        pltpu.async_copy(gidx_hbm.at[gbase + g], gibs[p], isems[p])
        pltpu.async_copy(didx_hbm.at[gbase + g], dibs[p], isems[p])

    def _idx_wait(p):
        pltpu.make_async_copy(gidx_hbm.at[0], gibs[p], isems[p]).wait()
        pltpu.make_async_copy(didx_hbm.at[0], dibs[p], isems[p]).wait()

    def _start_gather(p, b):
        pltpu.async_copy(xp_hbm.at[gibs[p].at[b]], bufs[b], gsems[b])

    def _wait_gather(p, b):
        pltpu.make_async_copy(xp_hbm.at[gibs[p].at[b]], bufs[b],
                              gsems[b]).wait()

    def _start_scatter(p, b):
        pltpu.async_copy(bufs[b], m_sh.at[dibs[p].at[b]], ssems[b], add=True)

    def _wait_scatter(p, b):
        pltpu.make_async_copy(bufs[b], m_sh.at[dibs[p].at[b]],
                              ssems[b]).wait()

    pltpu.sync_copy(zeros_hbm.at[pl.ds(s * _RPT, _RPT)],
                    m_sh.at[pl.ds(s * _RPT, _RPT)])

    _idx_load(0, 0)
    _idx_load(1, 1)
    plsc.subcore_barrier()
    _idx_wait(0)
    for b in range(_NBUF):
        _start_gather(0, b)

    def _do_group(p, q, prefetch_g):
        for b in range(_NBUF):
            _wait_gather(p, b)
            _start_scatter(p, b)
        _idx_wait(q)
        for b in range(_NBUF):
            _wait_scatter(p, b)
            _start_gather(q, b)
        _idx_load(prefetch_g, p)

    def _pair(i, carry):
        g = 2 * i
        _do_group(0, 1, g + 2)
        _do_group(1, 0, g + 3)
        return carry

    lax.fori_loop(0, npair, _pair, 0)
    for b in range(_NBUF):
        _wait_gather(0, b)
    _idx_wait(1)

    plsc.subcore_barrier()
    pltpu.sync_copy(m_sh.at[pl.ds(s * _RPT, _RPT)],
                    out_hbm.at[c, pl.ds(s * _RPT, _RPT)])


_mp_call = pl.kernel(
    _mp_body,
    out_type=jax.ShapeDtypeStruct((_NSC, _NPAD, H), jnp.float32),
    mesh=plsc.VectorSubcoreMesh(core_axis_name="c", subcore_axis_name="s"),
    scratch_types=[
        pltpu.VMEM_SHARED((_NPAD, H), jnp.float32),
        pltpu.VMEM((_NBUF, _CSZ), jnp.int32),
        pltpu.VMEM((_NBUF, _CSZ), jnp.int32),
        pltpu.VMEM((_NBUF, _CSZ), jnp.int32),
        pltpu.VMEM((_NBUF, _CSZ), jnp.int32),
        pltpu.VMEM((_CSZ, H), jnp.float32),
        pltpu.VMEM((_CSZ, H), jnp.float32),
        pltpu.VMEM((_CSZ, H), jnp.float32),
    ] + [pltpu.SemaphoreType.DMA] * 8,
)


def _enc_body(x_ref, w_ref, b_ref, table_ref, h_ref, xp_ref):
    h = x_ref[...] @ w_ref[...] + b_ref[...]
    h_ref[...] = h
    for a in range(VOCAB):
        xp_ref[a] = jnp.maximum(h + table_ref[a], 0.0) + EPS


def _encode(x, W_enc, b_enc, table):
    h, xp = pl.pallas_call(
        _enc_body,
        grid=(N // _BLK,),
        in_specs=[
            pl.BlockSpec((_BLK, H), lambda i: (i, 0)),
            pl.BlockSpec((H, H), lambda i: (0, 0)),
            pl.BlockSpec((1, H), lambda i: (0, 0)),
            pl.BlockSpec((VOCAB, H), lambda i: (0, 0)),
        ],
        out_specs=[
            pl.BlockSpec((_BLK, H), lambda i: (i, 0)),
            pl.BlockSpec((VOCAB, _BLK, H), lambda i: (0, i, 0)),
        ],
        out_shape=[
            jax.ShapeDtypeStruct((N, H), jnp.float32),
            jax.ShapeDtypeStruct((VOCAB, N, H), jnp.float32),
        ],
    )(x, W_enc, b_enc.reshape(1, H), table)
    return h, xp.reshape(VOCAB * N, H)


def _ln(t, s, b):
    mu = jnp.mean(t, axis=-1, keepdims=True)
    var = jnp.mean((t - mu) ** 2, axis=-1, keepdims=True)
    return (t - mu) / jnp.sqrt(var + 1e-5) * s + b


def _mlp_body(h2_ref, parts_ref, res_ref, w1_ref, b1_ref, s1_ref, bb1_ref,
              w2_ref, b2_ref, ns_ref, nb_ref, table_ref,
              h_ref, h2n_ref, xp_ref):
    t = (h2_ref[...] + parts_ref[0] + parts_ref[1]) @ w1_ref[...] + b1_ref[...]
    t = _ln(t, s1_ref[...], bb1_ref[...])
    t = jnp.maximum(t, 0.0)
    h = t @ w2_ref[...] + b2_ref[...] + res_ref[...]
    h_ref[...] = h
    h2n = jnp.maximum(_ln(h, ns_ref[...], nb_ref[...]), 0.0)
    h2n_ref[...] = h2n
    for a in range(VOCAB):
        xp_ref[a] = jnp.maximum(h2n + table_ref[a], 0.0) + EPS


def _mlp_x(h2, parts, res, W1l, b1l, s1l, bb1l, W2l, b2l, nsl, nbl, table):
    h, h2n, xp = pl.pallas_call(
        _mlp_body,
        grid=(N // _BLK,),
        in_specs=[
            pl.BlockSpec((_BLK, H), lambda i: (i, 0)),
            pl.BlockSpec((_NSC, _BLK, H), lambda i: (0, i, 0)),
            pl.BlockSpec((_BLK, H), lambda i: (i, 0)),
            pl.BlockSpec((H, 2 * H), lambda i: (0, 0)),
            pl.BlockSpec((1, 2 * H), lambda i: (0, 0)),
            pl.BlockSpec((1, 2 * H), lambda i: (0, 0)),
            pl.BlockSpec((1, 2 * H), lambda i: (0, 0)),
            pl.BlockSpec((2 * H, H), lambda i: (0, 0)),
            pl.BlockSpec((1, H), lambda i: (0, 0)),
            pl.BlockSpec((1, H), lambda i: (0, 0)),
            pl.BlockSpec((1, H), lambda i: (0, 0)),
            pl.BlockSpec((VOCAB, H), lambda i: (0, 0)),
        ],
        out_specs=[
            pl.BlockSpec((_BLK, H), lambda i: (i, 0)),
            pl.BlockSpec((_BLK, H), lambda i: (i, 0)),
            pl.BlockSpec((VOCAB, _BLK, H), lambda i: (0, i, 0)),
        ],
        out_shape=[
            jax.ShapeDtypeStruct((N, H), jnp.float32),
            jax.ShapeDtypeStruct((N, H), jnp.float32),
            jax.ShapeDtypeStruct((VOCAB, N, H), jnp.float32),
        ],
    )(h2, parts, res, W1l, b1l.reshape(1, -1), s1l.reshape(1, -1),
      bb1l.reshape(1, -1), W2l, b2l.reshape(1, -1), nsl.reshape(1, -1),
      nbl.reshape(1, -1), table)
    return h, h2n, xp.reshape(VOCAB * N, H)


def _fin_body(h2_ref, parts_ref, res_ref, w1_ref, b1_ref, s1_ref, bb1_ref,
              w2_ref, b2_ref, ns_ref, nb_ref, p_ref, wp_ref, bp_ref,
              out_ref, sums_ref, cnt_ref):
    i = pl.program_id(0)
    t = (h2_ref[...] + parts_ref[0] + parts_ref[1]) @ w1_ref[...] + b1_ref[...]
    t = _ln(t, s1_ref[...], bb1_ref[...])
    t = jnp.maximum(t, 0.0)
    h = t @ w2_ref[...] + b2_ref[...] + res_ref[...]
    hf = _ln(h, ns_ref[...], nb_ref[...])
    p = p_ref[...]
    psum = lax.dot_general(p, hf, (((0,), (0,)), ((), ())),
                           preferred_element_type=jnp.float32)
    pcnt = lax.dot_general(p, jnp.ones((_BLK, 1), jnp.float32),
                           (((0,), (0,)), ((), ())),
                           preferred_element_type=jnp.float32)

    @pl.when(i == 0)
    def _():
        sums_ref[...] = jnp.zeros_like(sums_ref)
        cnt_ref[...] = jnp.zeros_like(cnt_ref)

    sums_ref[...] += psum
    cnt_ref[...] += pcnt

    @pl.when(i == N // _BLK - 1)
    def _():
        hg = sums_ref[...] / jnp.maximum(cnt_ref[...], 1.0)
        out_ref[...] = jax.nn.sigmoid(hg @ wp_ref[...] + bp_ref[...])


def _final(h2, parts, res, W1l, b1l, s1l, bb1l, W2l, b2l, nsl, nbl, P,
           W_pred, b_pred):
    return pl.pallas_call(
        _fin_body,
        grid=(N // _BLK,),
        in_specs=[
            pl.BlockSpec((_BLK, H), lambda i: (i, 0)),
            pl.BlockSpec((_NSC, _BLK, H), lambda i: (0, i, 0)),
            pl.BlockSpec((_BLK, H), lambda i: (i, 0)),
            pl.BlockSpec((H, 2 * H), lambda i: (0, 0)),
            pl.BlockSpec((1, 2 * H), lambda i: (0, 0)),
            pl.BlockSpec((1, 2 * H), lambda i: (0, 0)),
            pl.BlockSpec((1, 2 * H), lambda i: (0, 0)),
            pl.BlockSpec((2 * H, H), lambda i: (0, 0)),
            pl.BlockSpec((1, H), lambda i: (0, 0)),
            pl.BlockSpec((1, H), lambda i: (0, 0)),
            pl.BlockSpec((1, H), lambda i: (0, 0)),
            pl.BlockSpec((_BLK, G), lambda i: (i, 0)),
            pl.BlockSpec((H, 1), lambda i: (0, 0)),
            pl.BlockSpec((1, 1), lambda i: (0, 0)),
        ],
        out_specs=pl.BlockSpec((G, 1), lambda i: (0, 0)),
        out_shape=jax.ShapeDtypeStruct((G, 1), jnp.float32),
        scratch_shapes=[
            pltpu.VMEM((G, H), jnp.float32),
            pltpu.VMEM((G, 1), jnp.float32),
        ],
    )(h2, parts, res, W1l, b1l.reshape(1, -1), s1l.reshape(1, -1),
      bb1l.reshape(1, -1), W2l, b2l.reshape(1, -1), nsl.reshape(1, -1),
      nbl.reshape(1, -1), P, W_pred, b_pred.reshape(1, 1))


def kernel(x, edge_index, edge_attr, batch, W_enc, b_enc, edge_table, W1, b1,
           ln1s, ln1b, W2, b2, norm_s, norm_b, W_pred, b_pred):
    src = edge_index[0]
    dst = edge_index[1]

    total = _NGT * _NBUF * _CSZ
    npads = total - E
    cidx = edge_attr.astype(jnp.int32) * N + src
    gpad = jnp.arange(npads, dtype=jnp.int32) % (VOCAB * N)
    gidx = jnp.concatenate([cidx, gpad]).reshape(_NGT, _NBUF, _CSZ)
    trash = N + (jnp.arange(npads, dtype=jnp.int32) % (_NPAD - N))
    didx = jnp.concatenate([dst, trash]).reshape(_NGT, _NBUF, _CSZ)
    zeros = jnp.zeros((_NPAD, H), jnp.float32)
    P = (batch[:, None] == jnp.arange(G, dtype=batch.dtype)[None, :]
         ).astype(jnp.float32)

    h, xp = _encode(x, W_enc, b_enc, edge_table)
    h2 = h
    res = jnp.zeros((N, H), jnp.float32)
    for l in range(L - 1):
        parts = _mp_call(xp, gidx, didx, zeros)
        h, h2, xp = _mlp_x(h2, parts, res, W1[l], b1[l], ln1s[l], ln1b[l],
                           W2[l], b2[l], norm_s[l], norm_b[l], edge_table)
        res = h
    parts = _mp_call(xp, gidx, didx, zeros)
    out = _final(h2, parts, res, W1[L - 1], b1[L - 1], ln1s[L - 1],
                 ln1b[L - 1], W2[L - 1], b2[L - 1], norm_s[L - 1],
                 norm_b[L - 1], P, W_pred, b_pred)
    return out.reshape(-1)


# speedup vs baseline: 5.5427x; 1.1725x over previous
import functools

import jax
import jax.numpy as jnp
from jax import lax
from jax.experimental import pallas as pl
from jax.experimental.pallas import tpu as pltpu
from jax.experimental.pallas import tpu_sc as plsc

N = 10000
E = 320000
H = 128
L = 7
G = 64
VOCAB = 8
EPS = 1e-7

_BLK = 1000

_NSC = 2
_NSUB = 16
_NW = _NSC * _NSUB
_CSZ = 120
_NBUF = 3
_NGRP0 = 28
_NGRP1 = 28
_NGT = _NSUB * (_NGRP0 + _NGRP1) + 2
_NPAD = 10112
_RPT = _NPAD // _NSUB


def _mp_body(xp_hbm, gidx_hbm, didx_hbm, zeros_hbm, out_hbm,
             m_sh, gib0, gib1, dib0, dib1, r0, r1, r2,
             gs0, gs1, gs2, ss0, ss1, ss2, is0, is1):
    c = lax.axis_index("c")
    s = lax.axis_index("s")
    wid = c * _NSUB + s
    gbase = jnp.where(c == 0, s * _NGRP0, _NSUB * _NGRP0 + s * _NGRP1)
    npair = jnp.where(c == 0, _NGRP0 // 2, _NGRP1 // 2)
    bufs = (r0, r1, r2)
    gsems = (gs0, gs1, gs2)
    ssems = (ss0, ss1, ss2)
    gibs = (gib0, gib1)
    dibs = (dib0, dib1)
    isems = (is0, is1)

    def _idx_load(g, p):
        pltpu.async_copy(gidx_hbm.at[gbase + g], gibs[p], isems[p])
        pltpu.async_copy(didx_hbm.at[gbase + g], dibs[p], isems[p])

    def _idx_wait(p):
        pltpu.make_async_copy(gidx_hbm.at[0], gibs[p], isems[p]).wait()
        pltpu.make_async_copy(didx_hbm.at[0], dibs[p], isems[p]).wait()

    def _start_gather(p, b):
        pltpu.async_copy(xp_hbm.at[gibs[p].at[b]], bufs[b], gsems[b])

    def _wait_gather(p, b):
        pltpu.make_async_copy(xp_hbm.at[gibs[p].at[b]], bufs[b],
                              gsems[b]).wait()

    def _start_scatter(p, b):
        pltpu.async_copy(bufs[b], m_sh.at[dibs[p].at[b]], ssems[b], add=True)

    def _wait_scatter(p, b):
        pltpu.make_async_copy(bufs[b], m_sh.at[dibs[p].at[b]],
                              ssems[b]).wait()

    pltpu.sync_copy(zeros_hbm.at[pl.ds(s * _RPT, _RPT)],
                    m_sh.at[pl.ds(s * _RPT, _RPT)])

    _idx_load(0, 0)
    _idx_load(1, 1)
    plsc.subcore_barrier()
    _idx_wait(0)
    for b in range(_NBUF):
        _start_gather(0, b)

    def _do_group(p, q, prefetch_g):
        for b in range(_NBUF):
            _wait_gather(p, b)
            _start_scatter(p, b)
        _idx_wait(q)
        for b in range(_NBUF):
            _wait_scatter(p, b)
            _start_gather(q, b)
        _idx_load(prefetch_g, p)

    def _pair(i, carry):
        g = 2 * i
        _do_group(0, 1, g + 2)
        _do_group(1, 0, g + 3)
        return carry

    lax.fori_loop(0, npair, _pair, 0)
    for b in range(_NBUF):
        _wait_gather(0, b)
    _idx_wait(1)

    plsc.subcore_barrier()
    pltpu.sync_copy(m_sh.at[pl.ds(s * _RPT, _RPT)],
                    out_hbm.at[c, pl.ds(s * _RPT, _RPT)])


_mp_call = pl.kernel(
    _mp_body,
    out_type=jax.ShapeDtypeStruct((_NSC, _NPAD, H), jnp.float32),
    mesh=plsc.VectorSubcoreMesh(core_axis_name="c", subcore_axis_name="s"),
    scratch_types=[
        pltpu.VMEM_SHARED((_NPAD, H), jnp.float32),
        pltpu.VMEM((_NBUF, _CSZ), jnp.int32),
        pltpu.VMEM((_NBUF, _CSZ), jnp.int32),
        pltpu.VMEM((_NBUF, _CSZ), jnp.int32),
        pltpu.VMEM((_NBUF, _CSZ), jnp.int32),
        pltpu.VMEM((_CSZ, H), jnp.float32),
        pltpu.VMEM((_CSZ, H), jnp.float32),
        pltpu.VMEM((_CSZ, H), jnp.float32),
    ] + [pltpu.SemaphoreType.DMA] * 8,
)


def _enc_body(x_ref, w_ref, b_ref, table_ref, h_ref, xp_ref):
    h = x_ref[...] @ w_ref[...] + b_ref[...]
    h_ref[...] = h
    for a in range(VOCAB):
        xp_ref[a] = jnp.maximum(h + table_ref[a], 0.0) + EPS


def _encode(x, W_enc, b_enc, table):
    h, xp = pl.pallas_call(
        _enc_body,
        grid=(N // _BLK,),
        in_specs=[
            pl.BlockSpec((_BLK, H), lambda i: (i, 0)),
            pl.BlockSpec((H, H), lambda i: (0, 0)),
            pl.BlockSpec((1, H), lambda i: (0, 0)),
            pl.BlockSpec((VOCAB, H), lambda i: (0, 0)),
        ],
        out_specs=[
            pl.BlockSpec((_BLK, H), lambda i: (i, 0)),
            pl.BlockSpec((VOCAB, _BLK, H), lambda i: (0, i, 0)),
        ],
        out_shape=[
            jax.ShapeDtypeStruct((N, H), jnp.float32),
            jax.ShapeDtypeStruct((VOCAB, N, H), jnp.float32),
        ],
    )(x, W_enc, b_enc.reshape(1, H), table)
    return h, xp.reshape(VOCAB * N, H)


def _ln(t, s, b):
    mu = jnp.mean(t, axis=-1, keepdims=True)
    var = jnp.mean((t - mu) ** 2, axis=-1, keepdims=True)
    return (t - mu) / jnp.sqrt(var + 1e-5) * s + b


def _mlp_body(h2_ref, parts_ref, res_ref, w1_ref, b1_ref, s1_ref, bb1_ref,
              w2_ref, b2_ref, ns_ref, nb_ref, table_ref,
              h_ref, h2n_ref, xp_ref):
    t = (h2_ref[...] + parts_ref[0] + parts_ref[1]) @ w1_ref[...] + b1_ref[...]
    t = _ln(t, s1_ref[...], bb1_ref[...])
    t = jnp.maximum(t, 0.0)
    h = t @ w2_ref[...] + b2_ref[...] + res_ref[...]
    h_ref[...] = h
    h2n = jnp.maximum(_ln(h, ns_ref[...], nb_ref[...]), 0.0)
    h2n_ref[...] = h2n
    for a in range(VOCAB):
        xp_ref[a] = jnp.maximum(h2n + table_ref[a], 0.0) + EPS


def _mlp_x(h2, parts, res, W1l, b1l, s1l, bb1l, W2l, b2l, nsl, nbl, table):
    h, h2n, xp = pl.pallas_call(
        _mlp_body,
        grid=(N // _BLK,),
        in_specs=[
            pl.BlockSpec((_BLK, H), lambda i: (i, 0)),
            pl.BlockSpec((_NSC, _BLK, H), lambda i: (0, i, 0)),
            pl.BlockSpec((_BLK, H), lambda i: (i, 0)),
            pl.BlockSpec((H, 2 * H), lambda i: (0, 0)),
            pl.BlockSpec((1, 2 * H), lambda i: (0, 0)),
            pl.BlockSpec((1, 2 * H), lambda i: (0, 0)),
            pl.BlockSpec((1, 2 * H), lambda i: (0, 0)),
            pl.BlockSpec((2 * H, H), lambda i: (0, 0)),
            pl.BlockSpec((1, H), lambda i: (0, 0)),
            pl.BlockSpec((1, H), lambda i: (0, 0)),
            pl.BlockSpec((1, H), lambda i: (0, 0)),
            pl.BlockSpec((VOCAB, H), lambda i: (0, 0)),
        ],
        out_specs=[
            pl.BlockSpec((_BLK, H), lambda i: (i, 0)),
            pl.BlockSpec((_BLK, H), lambda i: (i, 0)),
            pl.BlockSpec((VOCAB, _BLK, H), lambda i: (0, i, 0)),
        ],
        out_shape=[
            jax.ShapeDtypeStruct((N, H), jnp.float32),
            jax.ShapeDtypeStruct((N, H), jnp.float32),
            jax.ShapeDtypeStruct((VOCAB, N, H), jnp.float32),
        ],
    )(h2, parts, res, W1l, b1l.reshape(1, -1), s1l.reshape(1, -1),
      bb1l.reshape(1, -1), W2l, b2l.reshape(1, -1), nsl.reshape(1, -1),
      nbl.reshape(1, -1), table)
    return h, h2n, xp.reshape(VOCAB * N, H)


def _fin_body(h2_ref, parts_ref, res_ref, w1_ref, b1_ref, s1_ref, bb1_ref,
              w2_ref, b2_ref, ns_ref, nb_ref, p_ref, wp_ref, bp_ref,
              out_ref, sums_ref, cnt_ref):
    i = pl.program_id(0)
    t = (h2_ref[...] + parts_ref[0] + parts_ref[1]) @ w1_ref[...] + b1_ref[...]
    t = _ln(t, s1_ref[...], bb1_ref[...])
    t = jnp.maximum(t, 0.0)
    h = t @ w2_ref[...] + b2_ref[...] + res_ref[...]
    hf = _ln(h, ns_ref[...], nb_ref[...])
    p = p_ref[...]
    psum = lax.dot_general(p, hf, (((0,), (0,)), ((), ())),
                           preferred_element_type=jnp.float32)
    pcnt = lax.dot_general(p, jnp.ones((_BLK, 1), jnp.float32),
                           (((0,), (0,)), ((), ())),
                           preferred_element_type=jnp.float32)

    @pl.when(i == 0)
    def _():
        sums_ref[...] = jnp.zeros_like(sums_ref)
        cnt_ref[...] = jnp.zeros_like(cnt_ref)

    sums_ref[...] += psum
    cnt_ref[...] += pcnt

    @pl.when(i == N // _BLK - 1)
    def _():
        hg = sums_ref[...] / jnp.maximum(cnt_ref[...], 1.0)
        out_ref[...] = jax.nn.sigmoid(hg @ wp_ref[...] + bp_ref[...])


def _final(h2, parts, res, W1l, b1l, s1l, bb1l, W2l, b2l, nsl, nbl, P,
           W_pred, b_pred):
    return pl.pallas_call(
        _fin_body,
        grid=(N // _BLK,),
        in_specs=[
            pl.BlockSpec((_BLK, H), lambda i: (i, 0)),
            pl.BlockSpec((_NSC, _BLK, H), lambda i: (0, i, 0)),
            pl.BlockSpec((_BLK, H), lambda i: (i, 0)),
            pl.BlockSpec((H, 2 * H), lambda i: (0, 0)),
            pl.BlockSpec((1, 2 * H), lambda i: (0, 0)),
            pl.BlockSpec((1, 2 * H), lambda i: (0, 0)),
            pl.BlockSpec((1, 2 * H), lambda i: (0, 0)),
            pl.BlockSpec((2 * H, H), lambda i: (0, 0)),
            pl.BlockSpec((1, H), lambda i: (0, 0)),
            pl.BlockSpec((1, H), lambda i: (0, 0)),
            pl.BlockSpec((1, H), lambda i: (0, 0)),
            pl.BlockSpec((_BLK, G), lambda i: (i, 0)),
            pl.BlockSpec((H, 1), lambda i: (0, 0)),
            pl.BlockSpec((1, 1), lambda i: (0, 0)),
        ],
        out_specs=pl.BlockSpec((G, 1), lambda i: (0, 0)),
        out_shape=jax.ShapeDtypeStruct((G, 1), jnp.float32),
        scratch_shapes=[
            pltpu.VMEM((G, H), jnp.float32),
            pltpu.VMEM((G, 1), jnp.float32),
        ],
    )(h2, parts, res, W1l, b1l.reshape(1, -1), s1l.reshape(1, -1),
      bb1l.reshape(1, -1), W2l, b2l.reshape(1, -1), nsl.reshape(1, -1),
      nbl.reshape(1, -1), P, W_pred, b_pred.reshape(1, 1))


def kernel(x, edge_index, edge_attr, batch, W_enc, b_enc, edge_table, W1, b1,
           ln1s, ln1b, W2, b2, norm_s, norm_b, W_pred, b_pred):
    src = edge_index[0]
    dst = edge_index[1]

    total = _NGT * _NBUF * _CSZ
    npads = total - E
    cidx = edge_attr.astype(jnp.int32) * N + src
    gpad = jnp.arange(npads, dtype=jnp.int32) % (VOCAB * N)
    gidx = jnp.concatenate([cidx, gpad]).reshape(_NGT, _NBUF, _CSZ)
    trash = N + (jnp.arange(npads, dtype=jnp.int32) % (_NPAD - N))
    didx = jnp.concatenate([dst, trash]).reshape(_NGT, _NBUF, _CSZ)
    zeros = jnp.zeros((_NPAD, H), jnp.float32)
    P = (batch[:, None] == jnp.arange(G, dtype=batch.dtype)[None, :]
         ).astype(jnp.float32)

    h, xp = _encode(x, W_enc, b_enc, edge_table)
    h2 = h
    res = jnp.zeros((N, H), jnp.float32)
    for l in range(L - 1):
        parts = _mp_call(xp, gidx, didx, zeros)
        h, h2, xp = _mlp_x(h2, parts, res, W1[l], b1[l], ln1s[l], ln1b[l],
                           W2[l], b2[l], norm_s[l], norm_b[l], edge_table)
        res = h
    parts = _mp_call(xp, gidx, didx, zeros)
    out = _final(h2, parts, res, W1[L - 1], b1[L - 1], ln1s[L - 1],
                 ln1b[L - 1], W2[L - 1], b2[L - 1], norm_s[L - 1],
                 norm_b[L - 1], P, W_pred, b_pred)
    return out.reshape(-1)


# speedup vs baseline: 5.6513x; 1.0196x over previous
import functools

import jax
import jax.numpy as jnp
from jax import lax
from jax.experimental import pallas as pl
from jax.experimental.pallas import tpu as pltpu
from jax.experimental.pallas import tpu_sc as plsc

N = 10000
E = 320000
H = 128
L = 7
G = 64
VOCAB = 8
EPS = 1e-7

_BLK = 1000

_NSC = 2
_NSUB = 16
_NW = _NSC * _NSUB
_CSZ = 88
_NBUF = 4
_NGRP0 = 30
_NGRP1 = 30
_NGT = _NSUB * (_NGRP0 + _NGRP1) + 2
_NPAD = 10112
_RPT = _NPAD // _NSUB


def _mp_body(xp_hbm, gidx_hbm, didx_hbm, zeros_hbm, out_hbm,
             m_sh, gib0, gib1, dib0, dib1, r0, r1, r2, r3,
             gs0, gs1, gs2, gs3, ss0, ss1, ss2, ss3, is0, is1):
    c = lax.axis_index("c")
    s = lax.axis_index("s")
    wid = c * _NSUB + s
    gbase = jnp.where(c == 0, s * _NGRP0, _NSUB * _NGRP0 + s * _NGRP1)
    npair = jnp.where(c == 0, _NGRP0 // 2, _NGRP1 // 2)
    bufs = (r0, r1, r2, r3)
    gsems = (gs0, gs1, gs2, gs3)
    ssems = (ss0, ss1, ss2, ss3)
    gibs = (gib0, gib1)
    dibs = (dib0, dib1)
    isems = (is0, is1)

    def _idx_load(g, p):
        pltpu.async_copy(gidx_hbm.at[gbase + g], gibs[p], isems[p])
        pltpu.async_copy(didx_hbm.at[gbase + g], dibs[p], isems[p])

    def _idx_wait(p):
        pltpu.make_async_copy(gidx_hbm.at[0], gibs[p], isems[p]).wait()
        pltpu.make_async_copy(didx_hbm.at[0], dibs[p], isems[p]).wait()

    def _start_gather(p, b):
        pltpu.async_copy(xp_hbm.at[gibs[p].at[b]], bufs[b], gsems[b])

    def _wait_gather(p, b):
        pltpu.make_async_copy(xp_hbm.at[gibs[p].at[b]], bufs[b],
                              gsems[b]).wait()

    def _start_scatter(p, b):
        pltpu.async_copy(bufs[b], m_sh.at[dibs[p].at[b]], ssems[b], add=True)

    def _wait_scatter(p, b):
        pltpu.make_async_copy(bufs[b], m_sh.at[dibs[p].at[b]],
                              ssems[b]).wait()

    pltpu.sync_copy(zeros_hbm.at[pl.ds(s * _RPT, _RPT)],
                    m_sh.at[pl.ds(s * _RPT, _RPT)])

    _idx_load(0, 0)
    _idx_load(1, 1)
    plsc.subcore_barrier()
    _idx_wait(0)
    for b in range(_NBUF):
        _start_gather(0, b)

    def _do_group(p, q, prefetch_g):
        for b in range(_NBUF):
            _wait_gather(p, b)
            _start_scatter(p, b)
        _idx_wait(q)
        for b in range(_NBUF):
            _wait_scatter(p, b)
            _start_gather(q, b)
        _idx_load(prefetch_g, p)

    def _pair(i, carry):
        g = 2 * i
        _do_group(0, 1, g + 2)
        _do_group(1, 0, g + 3)
        return carry

    lax.fori_loop(0, npair, _pair, 0)
    for b in range(_NBUF):
        _wait_gather(0, b)
    _idx_wait(1)

    plsc.subcore_barrier()
    pltpu.sync_copy(m_sh.at[pl.ds(s * _RPT, _RPT)],
                    out_hbm.at[c, pl.ds(s * _RPT, _RPT)])


_mp_call = pl.kernel(
    _mp_body,
    out_type=jax.ShapeDtypeStruct((_NSC, _NPAD, H), jnp.float32),
    mesh=plsc.VectorSubcoreMesh(core_axis_name="c", subcore_axis_name="s"),
    scratch_types=[
        pltpu.VMEM_SHARED((_NPAD, H), jnp.float32),
        pltpu.VMEM((_NBUF, _CSZ), jnp.int32),
        pltpu.VMEM((_NBUF, _CSZ), jnp.int32),
        pltpu.VMEM((_NBUF, _CSZ), jnp.int32),
        pltpu.VMEM((_NBUF, _CSZ), jnp.int32),
        pltpu.VMEM((_CSZ, H), jnp.float32),
        pltpu.VMEM((_CSZ, H), jnp.float32),
        pltpu.VMEM((_CSZ, H), jnp.float32),
        pltpu.VMEM((_CSZ, H), jnp.float32),
    ] + [pltpu.SemaphoreType.DMA] * 10,
)


def _enc_body(x_ref, w_ref, b_ref, table_ref, h_ref, xp_ref):
    h = x_ref[...] @ w_ref[...] + b_ref[...]
    h_ref[...] = h
    for a in range(VOCAB):
        xp_ref[a] = jnp.maximum(h + table_ref[a], 0.0) + EPS


def _encode(x, W_enc, b_enc, table):
    h, xp = pl.pallas_call(
        _enc_body,
        grid=(N // _BLK,),
        in_specs=[
            pl.BlockSpec((_BLK, H), lambda i: (i, 0)),
            pl.BlockSpec((H, H), lambda i: (0, 0)),
            pl.BlockSpec((1, H), lambda i: (0, 0)),
            pl.BlockSpec((VOCAB, H), lambda i: (0, 0)),
        ],
        out_specs=[
            pl.BlockSpec((_BLK, H), lambda i: (i, 0)),
            pl.BlockSpec((VOCAB, _BLK, H), lambda i: (0, i, 0)),
        ],
        out_shape=[
            jax.ShapeDtypeStruct((N, H), jnp.float32),
            jax.ShapeDtypeStruct((VOCAB, N, H), jnp.float32),
        ],
    )(x, W_enc, b_enc.reshape(1, H), table)
    return h, xp.reshape(VOCAB * N, H)


def _ln(t, s, b):
    mu = jnp.mean(t, axis=-1, keepdims=True)
    var = jnp.mean((t - mu) ** 2, axis=-1, keepdims=True)
    return (t - mu) / jnp.sqrt(var + 1e-5) * s + b


def _mlp_body(h2_ref, parts_ref, res_ref, w1_ref, b1_ref, s1_ref, bb1_ref,
              w2_ref, b2_ref, ns_ref, nb_ref, table_ref,
              h_ref, h2n_ref, xp_ref):
    t = (h2_ref[...] + parts_ref[0] + parts_ref[1]) @ w1_ref[...] + b1_ref[...]
    t = _ln(t, s1_ref[...], bb1_ref[...])
    t = jnp.maximum(t, 0.0)
    h = t @ w2_ref[...] + b2_ref[...] + res_ref[...]
    h_ref[...] = h
    h2n = jnp.maximum(_ln(h, ns_ref[...], nb_ref[...]), 0.0)
    h2n_ref[...] = h2n
    for a in range(VOCAB):
        xp_ref[a] = jnp.maximum(h2n + table_ref[a], 0.0) + EPS


def _mlp_x(h2, parts, res, W1l, b1l, s1l, bb1l, W2l, b2l, nsl, nbl, table):
    h, h2n, xp = pl.pallas_call(
        _mlp_body,
        grid=(N // _BLK,),
        in_specs=[
            pl.BlockSpec((_BLK, H), lambda i: (i, 0)),
            pl.BlockSpec((_NSC, _BLK, H), lambda i: (0, i, 0)),
            pl.BlockSpec((_BLK, H), lambda i: (i, 0)),
            pl.BlockSpec((H, 2 * H), lambda i: (0, 0)),
            pl.BlockSpec((1, 2 * H), lambda i: (0, 0)),
            pl.BlockSpec((1, 2 * H), lambda i: (0, 0)),
            pl.BlockSpec((1, 2 * H), lambda i: (0, 0)),
            pl.BlockSpec((2 * H, H), lambda i: (0, 0)),
            pl.BlockSpec((1, H), lambda i: (0, 0)),
            pl.BlockSpec((1, H), lambda i: (0, 0)),
            pl.BlockSpec((1, H), lambda i: (0, 0)),
            pl.BlockSpec((VOCAB, H), lambda i: (0, 0)),
        ],
        out_specs=[
            pl.BlockSpec((_BLK, H), lambda i: (i, 0)),
            pl.BlockSpec((_BLK, H), lambda i: (i, 0)),
            pl.BlockSpec((VOCAB, _BLK, H), lambda i: (0, i, 0)),
        ],
        out_shape=[
            jax.ShapeDtypeStruct((N, H), jnp.float32),
            jax.ShapeDtypeStruct((N, H), jnp.float32),
            jax.ShapeDtypeStruct((VOCAB, N, H), jnp.float32),
        ],
    )(h2, parts, res, W1l, b1l.reshape(1, -1), s1l.reshape(1, -1),
      bb1l.reshape(1, -1), W2l, b2l.reshape(1, -1), nsl.reshape(1, -1),
      nbl.reshape(1, -1), table)
    return h, h2n, xp.reshape(VOCAB * N, H)


def _fin_body(h2_ref, parts_ref, res_ref, w1_ref, b1_ref, s1_ref, bb1_ref,
              w2_ref, b2_ref, ns_ref, nb_ref, p_ref, wp_ref, bp_ref,
              out_ref, sums_ref, cnt_ref):
    i = pl.program_id(0)
    t = (h2_ref[...] + parts_ref[0] + parts_ref[1]) @ w1_ref[...] + b1_ref[...]
    t = _ln(t, s1_ref[...], bb1_ref[...])
    t = jnp.maximum(t, 0.0)
    h = t @ w2_ref[...] + b2_ref[...] + res_ref[...]
    hf = _ln(h, ns_ref[...], nb_ref[...])
    p = p_ref[...]
    psum = lax.dot_general(p, hf, (((0,), (0,)), ((), ())),
                           preferred_element_type=jnp.float32)
    pcnt = lax.dot_general(p, jnp.ones((_BLK, 1), jnp.float32),
                           (((0,), (0,)), ((), ())),
                           preferred_element_type=jnp.float32)

    @pl.when(i == 0)
    def _():
        sums_ref[...] = jnp.zeros_like(sums_ref)
        cnt_ref[...] = jnp.zeros_like(cnt_ref)

    sums_ref[...] += psum
    cnt_ref[...] += pcnt

    @pl.when(i == N // _BLK - 1)
    def _():
        hg = sums_ref[...] / jnp.maximum(cnt_ref[...], 1.0)
        out_ref[...] = jax.nn.sigmoid(hg @ wp_ref[...] + bp_ref[...])


def _final(h2, parts, res, W1l, b1l, s1l, bb1l, W2l, b2l, nsl, nbl, P,
           W_pred, b_pred):
    return pl.pallas_call(
        _fin_body,
        grid=(N // _BLK,),
        in_specs=[
            pl.BlockSpec((_BLK, H), lambda i: (i, 0)),
            pl.BlockSpec((_NSC, _BLK, H), lambda i: (0, i, 0)),
            pl.BlockSpec((_BLK, H), lambda i: (i, 0)),
            pl.BlockSpec((H, 2 * H), lambda i: (0, 0)),
            pl.BlockSpec((1, 2 * H), lambda i: (0, 0)),
            pl.BlockSpec((1, 2 * H), lambda i: (0, 0)),
            pl.BlockSpec((1, 2 * H), lambda i: (0, 0)),
            pl.BlockSpec((2 * H, H), lambda i: (0, 0)),
            pl.BlockSpec((1, H), lambda i: (0, 0)),
            pl.BlockSpec((1, H), lambda i: (0, 0)),
            pl.BlockSpec((1, H), lambda i: (0, 0)),
            pl.BlockSpec((_BLK, G), lambda i: (i, 0)),
            pl.BlockSpec((H, 1), lambda i: (0, 0)),
            pl.BlockSpec((1, 1), lambda i: (0, 0)),
        ],
        out_specs=pl.BlockSpec((G, 1), lambda i: (0, 0)),
        out_shape=jax.ShapeDtypeStruct((G, 1), jnp.float32),
        scratch_shapes=[
            pltpu.VMEM((G, H), jnp.float32),
            pltpu.VMEM((G, 1), jnp.float32),
        ],
    )(h2, parts, res, W1l, b1l.reshape(1, -1), s1l.reshape(1, -1),
      bb1l.reshape(1, -1), W2l, b2l.reshape(1, -1), nsl.reshape(1, -1),
      nbl.reshape(1, -1), P, W_pred, b_pred.reshape(1, 1))


def kernel(x, edge_index, edge_attr, batch, W_enc, b_enc, edge_table, W1, b1,
           ln1s, ln1b, W2, b2, norm_s, norm_b, W_pred, b_pred):
    src = edge_index[0]
    dst = edge_index[1]

    total = _NGT * _NBUF * _CSZ
    npads = total - E
    cidx = edge_attr.astype(jnp.int32) * N + src
    gpad = jnp.arange(npads, dtype=jnp.int32) % (VOCAB * N)
    gidx = jnp.concatenate([cidx, gpad]).reshape(_NGT, _NBUF, _CSZ)
    trash = N + (jnp.arange(npads, dtype=jnp.int32) % (_NPAD - N))
    didx = jnp.concatenate([dst, trash]).reshape(_NGT, _NBUF, _CSZ)
    zeros = jnp.zeros((_NPAD, H), jnp.float32)
    P = (batch[:, None] == jnp.arange(G, dtype=batch.dtype)[None, :]
         ).astype(jnp.float32)

    h, xp = _encode(x, W_enc, b_enc, edge_table)
    h2 = h
    res = jnp.zeros((N, H), jnp.float32)
    for l in range(L - 1):
        parts = _mp_call(xp, gidx, didx, zeros)
        h, h2, xp = _mlp_x(h2, parts, res, W1[l], b1[l], ln1s[l], ln1b[l],
                           W2[l], b2[l], norm_s[l], norm_b[l], edge_table)
        res = h
    parts = _mp_call(xp, gidx, didx, zeros)
    out = _final(h2, parts, res, W1[L - 1], b1[L - 1], ln1s[L - 1],
                 ln1b[L - 1], W2[L - 1], b2[L - 1], norm_s[L - 1],
                 norm_b[L - 1], P, W_pred, b_pred)
    return out.reshape(-1)
